# Initial kernel scaffold; baseline (speedup 1.0000x reference)
import jax
import jax.numpy as jnp
from jax.experimental import pallas as pl


def kernel(input, W_in1, b_in1, W_in2, b_in2, gl0_W1, gl0_b1, gl0_W2, gl0_b2, gl1_W1, gl1_b1, gl1_W2, gl1_b2, gl2_W1, gl2_b1, gl2_W2, gl2_b2):
    raise NotImplementedError("write your pallas kernel here")


# speedup vs baseline: 4.3942x; 4.3942x over previous
import jax
import jax.numpy as jnp
import numpy as np
from jax.experimental import pallas as pl

_F = 128
_N = 13
_SRC = (0, 0, 0, 0, 1, 2, 3, 4, 5, 6, 7, 8,
        1, 2, 3, 4, 5, 6, 7, 8, 9, 10, 11, 12)
_DST = (1, 2, 3, 4, 5, 6, 7, 8, 9, 10, 11, 12,
        0, 0, 0, 0, 1, 2, 3, 4, 5, 6, 7, 8)

_COLS = (0, 1, 2, 3, 4, 6, 8, 10, 5, 7, 9, 11)
_CC = np.array([16 + 12 * k + _COLS[j] for j in range(12) for k in range(4)])
_JJ = np.repeat(np.arange(12), 4)
_KK = np.tile(np.arange(4), 12)


def _build_input_weights(W_in1, b_in1, W_in2, b_in2):
    Wobj = jnp.zeros((64, _F), dtype=jnp.float32).at[0:16, :].set(W_in1.T)
    Wj = jnp.zeros((64, 12, _F), dtype=jnp.float32)
    Wj = Wj.at[_CC, _JJ, :].set(W_in2.T[_KK, :])
    Wh = jnp.concatenate([Wobj, Wj.reshape(64, 12 * _F)], axis=1)
    bh = jnp.concatenate([b_in1, jnp.tile(b_in2, 12)]).reshape(1, _N * _F)
    return Wh, bh


def _elu(x):
    return jnp.where(x > 0, x, jnp.exp(x) - 1.0)


def _gnn_kernel(x_ref, wh_ref, bh_ref,
                w1a0, w1b0, w20, b10, b20,
                w1a1, w1b1, w21, b11, b21,
                w1a2, w1b2, w22, b12, b22,
                out_ref):
    x = x_ref[:]
    hf = jnp.dot(x, wh_ref[:], preferred_element_type=jnp.float32) + bh_ref[:]
    h = [hf[:, _F * n:_F * (n + 1)] for n in range(_N)]
    for (w1a, w1b, w2, b1, b2) in (
            (w1a0, w1b0, w20, b10, b20),
            (w1a1, w1b1, w21, b11, b21),
            (w1a2, w1b2, w22, b12, b22)):
        w1a_v, w1b_v, w2_v = w1a[:], w1b[:], w2[:]
        b1_v, b2_v = b1[:], b2[:]
        P = [jnp.dot(h[n], w1a_v, preferred_element_type=jnp.float32) + b1_v
             for n in range(_N)]
        Q = [jnp.dot(h[n], w1b_v, preferred_element_type=jnp.float32)
             for n in range(_N)]
        new_h = [None] * _N
        for s, d in zip(_SRC, _DST):
            m = _elu(P[d] + Q[s])
            m = _elu(jnp.dot(m, w2_v, preferred_element_type=jnp.float32)
                     + b2_v)
            new_h[d] = m if new_h[d] is None else jnp.maximum(new_h[d], m)
        h = new_h
    for n in range(_N):
        out_ref[:, _F * n:_F * (n + 1)] = h[n]


def kernel(input, W_in1, b_in1, W_in2, b_in2,
           gl0_W1, gl0_b1, gl0_W2, gl0_b2,
           gl1_W1, gl1_b1, gl1_W2, gl1_b2,
           gl2_W1, gl2_b1, gl2_W2, gl2_b2):
    B = input.shape[0]
    Wh, bh = _build_input_weights(W_in1, b_in1, W_in2, b_in2)
    layer_args = []
    for W1, b1, W2, b2 in ((gl0_W1, gl0_b1, gl0_W2, gl0_b2),
                           (gl1_W1, gl1_b1, gl1_W2, gl1_b2),
                           (gl2_W1, gl2_b1, gl2_W2, gl2_b2)):
        layer_args += [W1[:, :_F].T, W1[:, _F:].T, W2.T,
                       b1.reshape(1, _F), b2.reshape(1, _F)]

    bB = 512 if B % 512 == 0 else B
    grid = (B // bB,)
    full = lambda a: pl.BlockSpec(a.shape, lambda i: (0, 0))
    out = pl.pallas_call(
        _gnn_kernel,
        grid=grid,
        in_specs=[pl.BlockSpec((bB, 64), lambda i: (i, 0)),
                  full(Wh), full(bh)] + [full(a) for a in layer_args],
        out_specs=pl.BlockSpec((bB, _N * _F), lambda i: (i, 0)),
        out_shape=jax.ShapeDtypeStruct((B, _N * _F), jnp.float32),
    )(input, Wh, bh, *layer_args)
    return out.reshape(B, _N, _F)
